# baseline (speedup 1.0000x reference)
import jax
import jax.numpy as jnp
from jax.experimental import pallas as pl
from jax.experimental.pallas import tpu as pltpu

B, L, D = 16, 4096, 1024


G = 8
GS = B // G


def _take_last_body(slen_ref, x_ref, out_ref, rows, sems, osem):
    for b in range(B):
        row = slen_ref[b] - 1
        pltpu.make_async_copy(
            x_ref.at[b, pl.ds(row, 1)], rows.at[pl.ds(b, 1)], sems.at[b // GS]
        ).start()
    for g in range(G):
        pltpu.make_async_copy(
            x_ref.at[0, pl.ds(0, GS)], rows.at[pl.ds(g * GS, GS)], sems.at[g]
        ).wait()
        pltpu.make_async_copy(
            rows.at[pl.ds(g * GS, GS)], out_ref.at[pl.ds(g * GS, GS)], osem
        ).start()
    pltpu.make_async_copy(rows, out_ref, osem).wait()


_take_last = pl.pallas_call(
    _take_last_body,
    out_shape=jax.ShapeDtypeStruct((B, D), jnp.float32),
    in_specs=[
        pl.BlockSpec(memory_space=pltpu.SMEM),
        pl.BlockSpec(memory_space=pl.ANY),
    ],
    out_specs=pl.BlockSpec(memory_space=pl.ANY),
    scratch_shapes=[
        pltpu.VMEM((B, D), jnp.float32),
        pltpu.SemaphoreType.DMA((G,)),
        pltpu.SemaphoreType.DMA,
    ],
)


@jax.jit
def kernel(x, seq_len):
    return _take_last(seq_len, x)


# speedup vs baseline: 1.0446x; 1.0446x over previous
import jax
import jax.numpy as jnp
from jax.experimental import pallas as pl
from jax.experimental.pallas import tpu as pltpu

B, L, D = 16, 4096, 1024


G = 2
GS = B // G


def _take_last_body(slen_ref, x_ref, out_ref, rows, sems, osem):
    for b in range(B):
        row = slen_ref[b] - 1
        pltpu.make_async_copy(
            x_ref.at[b, pl.ds(row, 1)], rows.at[pl.ds(b, 1)], sems.at[b // GS]
        ).start()
    for g in range(G):
        pltpu.make_async_copy(
            x_ref.at[0, pl.ds(0, GS)], rows.at[pl.ds(g * GS, GS)], sems.at[g]
        ).wait()
        pltpu.make_async_copy(
            rows.at[pl.ds(g * GS, GS)], out_ref.at[pl.ds(g * GS, GS)], osem
        ).start()
    pltpu.make_async_copy(rows, out_ref, osem).wait()


_take_last = pl.pallas_call(
    _take_last_body,
    out_shape=jax.ShapeDtypeStruct((B, D), jnp.float32),
    in_specs=[
        pl.BlockSpec(memory_space=pltpu.SMEM),
        pl.BlockSpec(memory_space=pl.ANY),
    ],
    out_specs=pl.BlockSpec(memory_space=pl.ANY),
    scratch_shapes=[
        pltpu.VMEM((B, D), jnp.float32),
        pltpu.SemaphoreType.DMA((G,)),
        pltpu.SemaphoreType.DMA,
    ],
)


@jax.jit
def kernel(x, seq_len):
    return _take_last(seq_len, x)
